# Initial kernel scaffold; baseline (speedup 1.0000x reference)
import functools

import jax
import jax.numpy as jnp
from jax import lax
from jax.experimental import pallas as pl
from jax.experimental.pallas import tpu as pltpu
from jax.experimental.pallas import tpu_sc as plsc

N = 10000
E = 320000
DIN = 128
DOUT = 128

NC = 2
NS = 16
NW = NC * NS
CHUNK = 80
E_ALL = E + N + 240
NCHUNK = E_ALL // (NW * CHUNK)
ROWS_PER_SUB = 640

_mesh = plsc.VectorSubcoreMesh(core_axis_name="c", subcore_axis_name="s")


@functools.partial(
    pl.kernel,
    mesh=_mesh,
    out_type=jax.ShapeDtypeStruct((NC, N), jnp.float32),
    scratch_types=[
        pltpu.VMEM((NCHUNK, CHUNK), jnp.int32),
        pltpu.VMEM((NCHUNK, CHUNK), jnp.float32),
        pltpu.VMEM((ROWS_PER_SUB,), jnp.float32),
        pltpu.VMEM_SHARED((N,), jnp.float32),
    ],
)
def _deg_kernel(col_hbm, ew_hbm, deg_hbm, col_v, ew_v, z_v, acc):
    cid = lax.axis_index("c")
    sid = lax.axis_index("s")
    wid = cid * NS + sid

    pltpu.sync_copy(col_hbm.at[wid], col_v)
    pltpu.sync_copy(ew_hbm.at[wid], ew_v)

    @pl.loop(0, ROWS_PER_SUB // 16)
    def _(i):
        z_v[pl.ds(i * 16, 16)] = jnp.zeros((16,), jnp.float32)

    @pl.when(sid < NS - 1)
    def _():
        pltpu.sync_copy(z_v, acc.at[pl.ds(sid * ROWS_PER_SUB, ROWS_PER_SUB)])

    @pl.when(sid == NS - 1)
    def _():
        pltpu.sync_copy(z_v.at[pl.ds(0, 400)], acc.at[pl.ds(9600, 400)])

    plsc.subcore_barrier()

    @pl.loop(0, NCHUNK)
    def _(c):
        pltpu.sync_copy(ew_v.at[c], acc.at[col_v.at[c]], add=True)

    plsc.subcore_barrier()

    @pl.when(sid < NS - 1)
    def _():
        pltpu.sync_copy(acc.at[pl.ds(sid * ROWS_PER_SUB, ROWS_PER_SUB)],
                        deg_hbm.at[cid, pl.ds(sid * ROWS_PER_SUB, ROWS_PER_SUB)])

    @pl.when(sid == NS - 1)
    def _():
        pltpu.sync_copy(acc.at[pl.ds(9600, 400)], deg_hbm.at[cid, pl.ds(9600, 400)])


def _mm_body(x_ref, w_ref, degp_ref, xw_ref, dinv_ref):
    xw_ref[...] = jnp.dot(x_ref[...], w_ref[...],
                          preferred_element_type=jnp.float32)
    deg = degp_ref[0:1, :] + degp_ref[1:2, :]
    dinv_ref[...] = lax.rsqrt(deg)


def _mm_dinv(x, W, degp):
    return pl.pallas_call(
        _mm_body,
        grid=(10,),
        in_specs=[
            pl.BlockSpec((1000, DIN), lambda i: (i, 0)),
            pl.BlockSpec((DIN, DOUT), lambda i: (0, 0)),
            pl.BlockSpec((NC, N), lambda i: (0, 0)),
        ],
        out_specs=[
            pl.BlockSpec((1000, DOUT), lambda i: (i, 0)),
            pl.BlockSpec((1, N), lambda i: (0, 0)),
        ],
        out_shape=[
            jax.ShapeDtypeStruct((N, DOUT), jnp.float32),
            jax.ShapeDtypeStruct((1, N), jnp.float32),
        ],
    )(x, W, degp)


@functools.partial(
    pl.kernel,
    mesh=_mesh,
    out_type=jax.ShapeDtypeStruct((NC, N, DOUT), jnp.float32),
    scratch_types=[
        pltpu.VMEM((NCHUNK, CHUNK), jnp.int32),
        pltpu.VMEM((NCHUNK, CHUNK), jnp.int32),
        pltpu.VMEM((NCHUNK, CHUNK), jnp.float32),
        pltpu.VMEM((N,), jnp.float32),
        pltpu.VMEM((CHUNK, DOUT), jnp.float32),
        pltpu.VMEM((CHUNK,), jnp.float32),
        pltpu.VMEM_SHARED((N, DOUT), jnp.float32),
    ],
)
def _scatter_kernel(xw_hbm, dinv_hbm, row_hbm, col_hbm, ew_hbm, out_hbm,
                    row_v, col_v, ew_v, dinv_v, rows_v, s_v, acc):
    cid = lax.axis_index("c")
    sid = lax.axis_index("s")
    wid = cid * NS + sid

    pltpu.sync_copy(row_hbm.at[wid], row_v)
    pltpu.sync_copy(col_hbm.at[wid], col_v)
    pltpu.sync_copy(ew_hbm.at[wid], ew_v)
    pltpu.sync_copy(dinv_hbm.at[0], dinv_v)

    @pl.loop(0, CHUNK)
    def _(i):
        for j in range(DOUT // 16):
            rows_v[i, pl.ds(j * 16, 16)] = jnp.zeros((16,), jnp.float32)

    @pl.when(sid < NS - 1)
    def _():
        @pl.loop(0, ROWS_PER_SUB // CHUNK)
        def _(k):
            pltpu.sync_copy(rows_v,
                            acc.at[pl.ds(sid * ROWS_PER_SUB + k * CHUNK, CHUNK)])

    @pl.when(sid == NS - 1)
    def _():
        @pl.loop(0, 5)
        def _(k):
            pltpu.sync_copy(rows_v, acc.at[pl.ds(9600 + k * CHUNK, CHUNK)])

    plsc.subcore_barrier()

    @pl.loop(0, NCHUNK)
    def _(c):
        @pl.loop(0, CHUNK // 16)
        def _(g):
            ridx = row_v[c, pl.ds(g * 16, 16)]
            cidx = col_v[c, pl.ds(g * 16, 16)]
            dr = plsc.load_gather(dinv_v, [ridx])
            dc = plsc.load_gather(dinv_v, [cidx])
            s_v[pl.ds(g * 16, 16)] = ew_v[c, pl.ds(g * 16, 16)] * dr * dc

        pltpu.sync_copy(xw_hbm.at[row_v.at[c]], rows_v)

        @pl.loop(0, CHUNK)
        def _(i):
            sv = plsc.load_gather(s_v, [jnp.full((16,), i, jnp.int32)])
            for j in range(DOUT // 16):
                rows_v[i, pl.ds(j * 16, 16)] = rows_v[i, pl.ds(j * 16, 16)] * sv

        pltpu.sync_copy(rows_v, acc.at[col_v.at[c]], add=True)

    plsc.subcore_barrier()

    @pl.when(sid < NS - 1)
    def _():
        pltpu.sync_copy(acc.at[pl.ds(sid * ROWS_PER_SUB, ROWS_PER_SUB)],
                        out_hbm.at[cid, pl.ds(sid * ROWS_PER_SUB, ROWS_PER_SUB)])

    @pl.when(sid == NS - 1)
    def _():
        pltpu.sync_copy(acc.at[pl.ds(9600, 400)], out_hbm.at[cid, pl.ds(9600, 400)])


def _fin_body(p_ref, b_ref, o_ref):
    o_ref[...] = p_ref[0] + p_ref[1] + b_ref[...]


def _final(partials, b):
    return pl.pallas_call(
        _fin_body,
        grid=(10,),
        in_specs=[
            pl.BlockSpec((NC, 1000, DOUT), lambda i: (0, i, 0)),
            pl.BlockSpec((1, DOUT), lambda i: (0, 0)),
        ],
        out_specs=pl.BlockSpec((1000, DOUT), lambda i: (i, 0)),
        out_shape=jax.ShapeDtypeStruct((N, DOUT), jnp.float32),
    )(partials, b.reshape(1, DOUT))


@jax.jit
def kernel(x, edge_index, edge_attr, W, b):
    loop = jnp.arange(N, dtype=jnp.int32)
    pad_i = jnp.zeros((E_ALL - E - N,), jnp.int32)
    pad_f = jnp.zeros((E_ALL - E - N,), jnp.float32)
    row3 = jnp.concatenate([edge_index[0], loop, pad_i]).reshape(NW, NCHUNK, CHUNK)
    col3 = jnp.concatenate([edge_index[1], loop, pad_i]).reshape(NW, NCHUNK, CHUNK)
    ew3 = jnp.concatenate([edge_attr, jnp.ones((N,), jnp.float32), pad_f]
                          ).reshape(NW, NCHUNK, CHUNK)

    degp = _deg_kernel(col3, ew3)
    xw, dinv = _mm_dinv(x, W, degp)
    partials = _scatter_kernel(xw, dinv, row3, col3, ew3)
    out = _final(partials, b)
    return (out, edge_index, edge_attr)


# speedup vs baseline: 7.9419x; 7.9419x over previous
import dataclasses
import functools

import jax
import jax.numpy as jnp
from jax import lax
from jax.experimental import pallas as pl
from jax.experimental.pallas import tpu as pltpu
from jax.experimental.pallas import tpu_sc as plsc

N = 10000
E = 320000
DIN = 128
DOUT = 128

NC = 2
NS = 16
NW = NC * NS
CHUNK = 112
NCHUNK = 96
BLK = 8
NBLK = NCHUNK // BLK
E_ALL = NW * NCHUNK * CHUNK
ROWS_PER_SUB = 640
NPAD = NS * ROWS_PER_SUB

_mesh = plsc.VectorSubcoreMesh(core_axis_name="c", subcore_axis_name="s")

_cp = pltpu.CompilerParams()
if "needs_layout_passes" in pltpu.CompilerParams.__dataclass_fields__:
    _cp = dataclasses.replace(_cp, needs_layout_passes=False)


@functools.partial(
    pl.kernel,
    mesh=_mesh,
    out_type=jax.ShapeDtypeStruct((NC, NPAD), jnp.float32),
    compiler_params=_cp,
    scratch_types=[
        pltpu.VMEM((NCHUNK, CHUNK), jnp.int32),
        pltpu.VMEM((NCHUNK, CHUNK), jnp.float32),
        pltpu.VMEM((ROWS_PER_SUB,), jnp.float32),
        pltpu.VMEM_SHARED((NPAD,), jnp.float32),
    ],
)
def _deg_kernel(col_hbm, ew_hbm, deg_hbm, col_v, ew_v, z_v, acc):
    cid = lax.axis_index("c")
    sid = lax.axis_index("s")
    wid = cid * NS + sid

    pltpu.sync_copy(col_hbm.at[wid], col_v)
    pltpu.sync_copy(ew_hbm.at[wid], ew_v)

    @pl.loop(0, ROWS_PER_SUB // 16)
    def _(i):
        z_v[pl.ds(i * 16, 16)] = jnp.zeros((16,), jnp.float32)

    pltpu.sync_copy(z_v, acc.at[pl.ds(sid * ROWS_PER_SUB, ROWS_PER_SUB)])

    plsc.subcore_barrier()

    @pl.loop(0, NCHUNK)
    def _(c):
        pltpu.sync_copy(ew_v.at[c], acc.at[col_v.at[c]], add=True)

    plsc.subcore_barrier()

    pltpu.sync_copy(acc.at[pl.ds(sid * ROWS_PER_SUB, ROWS_PER_SUB)],
                    deg_hbm.at[cid, pl.ds(sid * ROWS_PER_SUB, ROWS_PER_SUB)])


def _mm_body(x_ref, w_ref, degp_ref, xw_ref, dinv_ref):
    xw_ref[...] = jnp.dot(x_ref[...], w_ref[...],
                          preferred_element_type=jnp.float32)
    deg = degp_ref[0:1, :] + degp_ref[1:2, :]
    dinv_ref[...] = lax.rsqrt(deg)


def _mm_dinv(x, W, degp):
    return pl.pallas_call(
        _mm_body,
        grid=(10,),
        in_specs=[
            pl.BlockSpec((1000, DIN), lambda i: (i, 0)),
            pl.BlockSpec((DIN, DOUT), lambda i: (0, 0)),
            pl.BlockSpec((NC, NPAD), lambda i: (0, 0)),
        ],
        out_specs=[
            pl.BlockSpec((1000, DOUT), lambda i: (i, 0)),
            pl.BlockSpec((1, NPAD), lambda i: (0, 0)),
        ],
        out_shape=[
            jax.ShapeDtypeStruct((N, DOUT), jnp.float32),
            jax.ShapeDtypeStruct((1, NPAD), jnp.float32),
        ],
    )(x, W, degp)


@functools.partial(
    pl.kernel,
    mesh=_mesh,
    out_type=jax.ShapeDtypeStruct((NC, NPAD, DOUT), jnp.float32),
    compiler_params=_cp,
    scratch_types=[
        pltpu.VMEM((BLK, CHUNK), jnp.int32),
        pltpu.VMEM((BLK, CHUNK), jnp.int32),
        pltpu.VMEM((BLK, CHUNK), jnp.float32),
        pltpu.VMEM((NPAD,), jnp.float32),
        pltpu.VMEM((CHUNK, DOUT), jnp.float32),
        pltpu.VMEM((CHUNK,), jnp.float32),
        pltpu.VMEM_SHARED((NPAD, DOUT), jnp.float32),
    ],
)
def _scatter_kernel(xw_hbm, dinv_hbm, row_hbm, col_hbm, ew_hbm, out_hbm,
                    rowb, colb, ewb, dinv_v, rows_v, s_v, acc):
    cid = lax.axis_index("c")
    sid = lax.axis_index("s")
    wid = cid * NS + sid

    pltpu.sync_copy(dinv_hbm.at[0], dinv_v)

    @pl.loop(0, CHUNK)
    def _(i):
        for j in range(DOUT // 16):
            rows_v[i, pl.ds(j * 16, 16)] = jnp.zeros((16,), jnp.float32)

    @pl.loop(0, ROWS_PER_SUB // 80)
    def _(k):
        pltpu.sync_copy(rows_v.at[pl.ds(0, 80)],
                        acc.at[pl.ds(sid * ROWS_PER_SUB + k * 80, 80)])

    plsc.subcore_barrier()

    @pl.loop(0, NBLK)
    def _(blk):
        pltpu.sync_copy(row_hbm.at[wid, pl.ds(blk * BLK, BLK)], rowb)
        pltpu.sync_copy(col_hbm.at[wid, pl.ds(blk * BLK, BLK)], colb)
        pltpu.sync_copy(ew_hbm.at[wid, pl.ds(blk * BLK, BLK)], ewb)

        @pl.loop(0, BLK)
        def _(c):
            @pl.loop(0, CHUNK // 16)
            def _(g):
                ridx = rowb[c, pl.ds(g * 16, 16)]
                cidx = colb[c, pl.ds(g * 16, 16)]
                dr = plsc.load_gather(dinv_v, [ridx])
                dc = plsc.load_gather(dinv_v, [cidx])
                s_v[pl.ds(g * 16, 16)] = ewb[c, pl.ds(g * 16, 16)] * dr * dc

            pltpu.sync_copy(xw_hbm.at[rowb.at[c]], rows_v)

            @pl.loop(0, CHUNK)
            def _(i):
                sv = plsc.load_gather(s_v, [jnp.full((16,), i, jnp.int32)])
                for j in range(DOUT // 16):
                    rows_v[i, pl.ds(j * 16, 16)] = rows_v[i, pl.ds(j * 16, 16)] * sv

            pltpu.sync_copy(rows_v, acc.at[colb.at[c]], add=True)

    plsc.subcore_barrier()

    pltpu.sync_copy(acc.at[pl.ds(sid * ROWS_PER_SUB, ROWS_PER_SUB)],
                    out_hbm.at[cid, pl.ds(sid * ROWS_PER_SUB, ROWS_PER_SUB)])


def _fin_body(p_ref, b_ref, o_ref):
    o_ref[...] = p_ref[0] + p_ref[1] + b_ref[...]


def _final(partials, b):
    return pl.pallas_call(
        _fin_body,
        grid=(10,),
        in_specs=[
            pl.BlockSpec((NC, 1000, DOUT), lambda i: (0, i, 0)),
            pl.BlockSpec((1, DOUT), lambda i: (0, 0)),
        ],
        out_specs=pl.BlockSpec((1000, DOUT), lambda i: (i, 0)),
        out_shape=jax.ShapeDtypeStruct((N, DOUT), jnp.float32),
    )(partials, b.reshape(1, DOUT))


@jax.jit
def kernel(x, edge_index, edge_attr, W, b):
    npad_e = E_ALL - E - N
    loop = jnp.arange(N, dtype=jnp.int32)
    pad_i = jnp.zeros((npad_e,), jnp.int32)
    pad_f = jnp.zeros((npad_e,), jnp.float32)
    row3 = jnp.concatenate([edge_index[0], loop, pad_i]).reshape(NW, NCHUNK, CHUNK)
    col3 = jnp.concatenate([edge_index[1], loop, pad_i]).reshape(NW, NCHUNK, CHUNK)
    ew3 = jnp.concatenate([edge_attr, jnp.ones((N,), jnp.float32), pad_f]
                          ).reshape(NW, NCHUNK, CHUNK)

    degp = _deg_kernel(col3, ew3)
    xw, dinv = _mm_dinv(x, W, degp)
    partials = _scatter_kernel(xw, dinv, row3, col3, ew3)
    out = _final(partials, b)
    return (out, edge_index, edge_attr)
